# baseline (speedup 1.0000x reference)
import functools

import jax
import jax.numpy as jnp
from jax import lax
from jax.experimental import pallas as pl
from jax.experimental.pallas import tpu as pltpu
from jax.experimental.pallas import tpu_sc as plsc

N = 65536
ROW = 100
NC = 2
NS = 16
NW = NC * NS
ROWS_PER_W = N // NW
CHUNK_ROWS = 256
CHUNKS = ROWS_PER_W // CHUNK_ROWS
GROUPS = CHUNK_ROWS // 16
CHUNK_ELEMS = CHUNK_ROWS * ROW


def _ci(v):
    return lax.full((16,), v, jnp.int32)


def _cf(v):
    return lax.full((16,), v, jnp.float32)


@functools.partial(
    pl.kernel,
    out_type=jax.ShapeDtypeStruct((N * ROW,), jnp.float32),
    mesh=plsc.VectorSubcoreMesh(core_axis_name="c", subcore_axis_name="s"),
    scratch_types=[pltpu.VMEM((CHUNK_ELEMS,), jnp.float32)],
    compiler_params=pltpu.CompilerParams(needs_layout_passes=False),
)
def _solve(x_hbm, out_hbm, buf):
    wid = lax.axis_index("s") * NC + lax.axis_index("c")
    base = wid * (ROWS_PER_W * ROW)
    lanes = lax.iota(jnp.int32, 16) * ROW

    def chunk_body(ci, carry):
        off = base + ci * CHUNK_ELEMS
        pltpu.sync_copy(x_hbm.at[pl.ds(off, CHUNK_ELEMS)], buf)

        def group_body(gi, c2):
            idx0 = lanes + gi * (16 * ROW)
            orall = _ci(0)
            for l in range(10):
                gbase = idx0 + (l * 10)
                m = plsc.load_gather(buf, [gbase])
                b = _ci(1)
                for v in range(1, 10):
                    xv = plsc.load_gather(buf, [gbase + v])
                    gt = xv > m
                    b = jnp.where(gt, _ci(1 << v), b)
                    m = jnp.where(gt, xv, m)
                orall = orall | b
            one = _cf(1.0)
            zero = _cf(0.0)
            for l in range(1, 10):
                miss = (orall & _ci(1 << (l - 1))) == _ci(0)
                for v in range(10):
                    val = one if v == (l - 1) else zero
                    plsc.store_scatter(buf, [idx0 + (l * 10 + v)], val,
                                       mask=miss)
            return c2

        lax.fori_loop(0, GROUPS, group_body, 0)
        pltpu.sync_copy(buf, out_hbm.at[pl.ds(off, CHUNK_ELEMS)])
        return carry

    lax.fori_loop(0, CHUNKS, chunk_body, 0)


def kernel(memory_vb):
    x = memory_vb.reshape(-1)
    out = _solve(x)
    return out.reshape(memory_vb.shape)


# speedup vs baseline: 3.2538x; 3.2538x over previous
import functools

import jax
import jax.numpy as jnp
from jax import lax
from jax.experimental import pallas as pl
from jax.experimental.pallas import tpu as pltpu
from jax.experimental.pallas import tpu_sc as plsc

N = 65536
L = 10
V = 10
NC = 2
NS = 16
NW = NC * NS
ROWS_PER_W = N // NW
CHUNK_ROWS = 256
CHUNKS = ROWS_PER_W // CHUNK_ROWS
GROUPS = CHUNK_ROWS // 16


def _ci(v):
    return lax.full((16,), v, jnp.int32)


def _cf(v):
    return lax.full((16,), v, jnp.float32)


@functools.partial(
    pl.kernel,
    out_type=jax.ShapeDtypeStruct((N, L * V), jnp.float32),
    mesh=plsc.VectorSubcoreMesh(core_axis_name="c", subcore_axis_name="s"),
    scratch_types=[pltpu.VMEM((CHUNK_ROWS, L * V), jnp.float32)],
    compiler_params=pltpu.CompilerParams(needs_layout_passes=False),
)
def _solve(x_hbm, out_hbm, buf):
    wid = lax.axis_index("s") * NC + lax.axis_index("c")
    base_row = wid * ROWS_PER_W
    lanes = lax.iota(jnp.int32, 16)

    def chunk_body(ci, carry):
        row0 = base_row + ci * CHUNK_ROWS
        pltpu.sync_copy(x_hbm.at[pl.ds(row0, CHUNK_ROWS)], buf)

        def group_body(gi, c2):
            ridx = lanes + gi * 16
            orall = _ci(0)
            for l in range(L):
                m = plsc.load_gather(buf, [ridx, _ci(l * 10)])
                b = _ci(1)
                for v in range(1, V):
                    xv = plsc.load_gather(buf, [ridx, _ci(l * 10 + v)])
                    gt = xv > m
                    b = jnp.where(gt, _ci(1 << v), b)
                    m = jnp.where(gt, xv, m)
                orall = orall | b
            one = _cf(1.0)
            zero = _cf(0.0)
            for l in range(1, L):
                miss = (orall & _ci(1 << (l - 1))) == _ci(0)
                for v in range(V):
                    val = one if v == (l - 1) else zero
                    plsc.store_scatter(buf, [ridx, _ci(l * 10 + v)], val,
                                       mask=miss)
            return c2

        lax.fori_loop(0, GROUPS, group_body, 0)
        pltpu.sync_copy(buf, out_hbm.at[pl.ds(row0, CHUNK_ROWS)])
        return carry

    lax.fori_loop(0, CHUNKS, chunk_body, 0)


def kernel(memory_vb):
    out = _solve(memory_vb.reshape(N, L * V))
    return out.reshape(memory_vb.shape)
